# Initial kernel scaffold; baseline (speedup 1.0000x reference)
import jax
import jax.numpy as jnp
from jax.experimental import pallas as pl


def kernel(x, edge_index_dim0, edge_index_dim1, batch, W1_d0, W1_d1, W1_out, b1, W2_d0, W2_d1, W2_out, b2, F1, f1b, F2, f2b, F3, f3b):
    raise NotImplementedError("write your pallas kernel here")


# speedup vs baseline: 5.1178x; 5.1178x over previous
import jax
import jax.numpy as jnp
from jax import lax
from jax.experimental import pallas as pl
from jax.experimental.pallas import tpu as pltpu
from jax.experimental.pallas import tpu_sc as plsc

N = 10000
E = 320000
D = 128
G = 64

_NS = 16
_EPT = E // _NS
_C = 128
_NCH = _EPT // _C
_TAIL = _EPT - _NCH * _C
_RPT = 624
_RREM = N - _RPT * _NS

_BLKS = [(0, 128), (128, 128), (256, 128), (384, 128), (496, 128)]


def _make_sc_scatter():
    mesh = plsc.VectorSubcoreMesh(core_axis_name="c", subcore_axis_name="s")
    out_type = [jax.ShapeDtypeStruct((2, N, D), jnp.float32)]
    scratch = [
        pltpu.VMEM((_C,), jnp.int32),
        pltpu.VMEM((_C,), jnp.int32),
        pltpu.VMEM((_C, D), jnp.float32),
        pltpu.VMEM((_TAIL,), jnp.int32),
        pltpu.VMEM((_TAIL,), jnp.int32),
        pltpu.VMEM((_TAIL, D), jnp.float32),
        pltpu.VMEM_SHARED((N, D), jnp.float32),
    ]

    def body(h_hbm, ei_hbm, acc_out, idx_s, idx_d, rows, tidx_s, tidx_d, trows, acc):
        c = lax.axis_index("c")
        s = lax.axis_index("s")
        r0 = s * _RPT
        rr = _RPT * _NS

        def zrow(r, carry):
            def zcol(k, carry2):
                rows[r, pl.ds(k * 16, 16)] = jnp.zeros((16,), jnp.float32)
                return carry2
            lax.fori_loop(0, D // 16, zcol, None)
            return carry

        lax.fori_loop(0, _C, zrow, None)

        for boff, blen in _BLKS:
            pltpu.sync_copy(rows.at[pl.ds(0, blen)],
                            acc.at[pl.ds(r0 + boff, blen)])

        @pl.when(s == _NS - 1)
        def _():
            pltpu.sync_copy(rows.at[pl.ds(0, _RREM)], acc.at[pl.ds(rr, _RREM)])

        plsc.subcore_barrier()

        sbase = c * (2 * E) + s * _EPT
        dbase = sbase + E

        def chunk(j, _):
            off = j * _C
            pltpu.sync_copy(ei_hbm.at[pl.ds(sbase + off, _C)], idx_s)
            pltpu.sync_copy(ei_hbm.at[pl.ds(dbase + off, _C)], idx_d)
            pltpu.sync_copy(h_hbm.at[idx_s], rows)
            pltpu.sync_copy(rows, acc.at[idx_d], add=True)
            return _

        lax.fori_loop(0, _NCH, chunk, None)
        toff = _NCH * _C
        pltpu.sync_copy(ei_hbm.at[pl.ds(sbase + toff, _TAIL)], tidx_s)
        pltpu.sync_copy(ei_hbm.at[pl.ds(dbase + toff, _TAIL)], tidx_d)
        pltpu.sync_copy(h_hbm.at[tidx_s], trows)
        pltpu.sync_copy(trows, acc.at[tidx_d], add=True)

        plsc.subcore_barrier()
        for boff, blen in _BLKS:
            pltpu.sync_copy(acc.at[pl.ds(r0 + boff, blen)],
                            rows.at[pl.ds(0, blen)])
            pltpu.sync_copy(rows.at[pl.ds(0, blen)],
                            acc_out.at[c, pl.ds(r0 + boff, blen)])

        @pl.when(s == _NS - 1)
        def _():
            pltpu.sync_copy(acc.at[pl.ds(rr, _RREM)], trows.at[pl.ds(0, _RREM)])
            pltpu.sync_copy(trows.at[pl.ds(0, _RREM)],
                            acc_out.at[c, pl.ds(rr, _RREM)])

    return pl.kernel(body, out_type=out_type, mesh=mesh, scratch_types=scratch)


def _make_sc_deg():
    mesh = plsc.VectorSubcoreMesh(core_axis_name="c", subcore_axis_name="s")
    out_type = [jax.ShapeDtypeStruct((2, N, D), jnp.float32)]
    scratch = [
        pltpu.VMEM((_C,), jnp.int32),
        pltpu.VMEM((_TAIL,), jnp.int32),
        pltpu.VMEM((_C, D), jnp.float32),
        pltpu.VMEM_SHARED((N, D), jnp.float32),
    ]

    def body(ei_hbm, deg_out, idx_d, tidx_d, ones_v, degacc):
        c = lax.axis_index("c")
        s = lax.axis_index("s")
        r0 = s * _RPT
        rr = _RPT * _NS

        def fill(val):
            def frow(r, carry):
                def fcol(k, carry2):
                    ones_v[r, pl.ds(k * 16, 16)] = jnp.full((16,), val, jnp.float32)
                    return carry2
                lax.fori_loop(0, D // 16, fcol, None)
                return carry
            lax.fori_loop(0, _C, frow, None)

        fill(0.0)
        for boff, blen in _BLKS:
            pltpu.sync_copy(ones_v.at[pl.ds(0, blen)],
                            degacc.at[pl.ds(r0 + boff, blen)])

        @pl.when(s == _NS - 1)
        def _():
            pltpu.sync_copy(ones_v.at[pl.ds(0, _RREM)], degacc.at[pl.ds(rr, _RREM)])

        fill(1.0)
        plsc.subcore_barrier()

        dbase = c * (2 * E) + E + s * _EPT

        def chunk(j, _):
            pltpu.sync_copy(ei_hbm.at[pl.ds(dbase + j * _C, _C)], idx_d)
            pltpu.sync_copy(ones_v, degacc.at[idx_d], add=True)
            return _

        lax.fori_loop(0, _NCH, chunk, None)
        pltpu.sync_copy(ei_hbm.at[pl.ds(dbase + _NCH * _C, _TAIL)], tidx_d)
        pltpu.sync_copy(ones_v.at[pl.ds(0, _TAIL)], degacc.at[tidx_d], add=True)

        plsc.subcore_barrier()
        for boff, blen in _BLKS:
            pltpu.sync_copy(degacc.at[pl.ds(r0 + boff, blen)],
                            ones_v.at[pl.ds(0, blen)])
            pltpu.sync_copy(ones_v.at[pl.ds(0, blen)],
                            deg_out.at[c, pl.ds(r0 + boff, blen)])

        @pl.when(s == _NS - 1)
        def _():
            pltpu.sync_copy(degacc.at[pl.ds(rr, _RREM)], ones_v.at[pl.ds(0, _RREM)])
            pltpu.sync_copy(ones_v.at[pl.ds(0, _RREM)],
                            deg_out.at[c, pl.ds(rr, _RREM)])

    return pl.kernel(body, out_type=out_type, mesh=mesh, scratch_types=scratch)


_sc_scatter = _make_sc_scatter()
_sc_deg = _make_sc_deg()


_R = 1000
_NB = N // _R


def _tc_conv1_body(a_ref, deg_ref, w0_ref, w1_ref, wout_ref, b_ref, out_ref):
    f32 = jnp.float32
    wc0 = jnp.dot(w0_ref[...], wout_ref[...], preferred_element_type=f32)
    wc1 = jnp.dot(w1_ref[...], wout_ref[...], preferred_element_type=f32)
    r0 = 1.0 / jnp.maximum(deg_ref[0][:, :1], 1.0)
    r1 = 1.0 / jnp.maximum(deg_ref[1][:, :1], 1.0)
    out_ref[...] = (jnp.dot(a_ref[0] * r0, wc0, preferred_element_type=f32)
                    + jnp.dot(a_ref[1] * r1, wc1, preferred_element_type=f32)
                    + b_ref[...])


def _tc_conv1(a, deg, w0, w1, wout, b):
    return pl.pallas_call(
        _tc_conv1_body,
        grid=(_NB,),
        in_specs=[
            pl.BlockSpec((2, _R, D), lambda i: (0, i, 0)),
            pl.BlockSpec((2, _R, D), lambda i: (0, i, 0)),
            pl.BlockSpec((D, D), lambda i: (0, 0)),
            pl.BlockSpec((D, D), lambda i: (0, 0)),
            pl.BlockSpec((D, D), lambda i: (0, 0)),
            pl.BlockSpec((1, D), lambda i: (0, 0)),
        ],
        out_specs=pl.BlockSpec((_R, D), lambda i: (i, 0)),
        out_shape=jax.ShapeDtypeStruct((N, D), jnp.float32),
    )(a, deg, w0, w1, wout, b)


def _leaky(v):
    return jnp.where(v > 0, v, 0.01 * v)


def _tc_conv2_body(a_ref, deg_ref, w0_ref, w1_ref, wout_ref, b_ref, batch_ref,
                   f1_ref, f1b_ref, f2_ref, f2b_ref, f3_ref, f3b_ref,
                   out_ref, pooled_s, cnt_s):
    f32 = jnp.float32
    i = pl.program_id(0)
    wc0 = jnp.dot(w0_ref[...], wout_ref[...], preferred_element_type=f32)
    wc1 = jnp.dot(w1_ref[...], wout_ref[...], preferred_element_type=f32)
    r0 = 1.0 / jnp.maximum(deg_ref[0][:, :1], 1.0)
    r1 = 1.0 / jnp.maximum(deg_ref[1][:, :1], 1.0)
    h = (jnp.dot(a_ref[0] * r0, wc0, preferred_element_type=f32)
         + jnp.dot(a_ref[1] * r1, wc1, preferred_element_type=f32)
         + b_ref[...])
    oh = (batch_ref[...] == lax.broadcasted_iota(jnp.int32, (_R, G), 1)).astype(f32)
    dnums = (((0,), (0,)), ((), ()))
    pb = lax.dot_general(oh, h, dnums, preferred_element_type=f32)
    cb = lax.dot_general(oh, jnp.ones_like(h), dnums, preferred_element_type=f32)

    @pl.when(i == 0)
    def _():
        pooled_s[...] = pb
        cnt_s[...] = cb

    @pl.when(i > 0)
    def _():
        pooled_s[...] += pb
        cnt_s[...] += cb

    @pl.when(i == _NB - 1)
    def _():
        ge = pooled_s[...] / jnp.maximum(cnt_s[...], 1.0)
        z = _leaky(jnp.dot(ge, f1_ref[...], preferred_element_type=f32) + f1b_ref[...])
        z = _leaky(jnp.dot(z, f2_ref[...], preferred_element_type=f32) + f2b_ref[...])
        out_ref[...] = jnp.dot(z, f3_ref[...], preferred_element_type=f32) + f3b_ref[...]


def _tc_conv2(a, deg, w0, w1, wout, b, batch2, f1, f1b, f2, f2b, f3, f3b):
    full = lambda shape: pl.BlockSpec(shape, lambda i: tuple(0 for _ in shape))
    return pl.pallas_call(
        _tc_conv2_body,
        grid=(_NB,),
        in_specs=[
            pl.BlockSpec((2, _R, D), lambda i: (0, i, 0)),
            pl.BlockSpec((2, _R, D), lambda i: (0, i, 0)),
            full((D, D)), full((D, D)), full((D, D)), full((1, D)),
            pl.BlockSpec((_R, 1), lambda i: (i, 0)),
            full((D, 84)), full((1, 84)),
            full((84, 42)), full((1, 42)),
            full((42, D)), full((1, D)),
        ],
        out_specs=pl.BlockSpec((G, D), lambda i: (0, 0)),
        out_shape=jax.ShapeDtypeStruct((G, D), jnp.float32),
        scratch_shapes=[
            pltpu.VMEM((G, D), jnp.float32),
            pltpu.VMEM((G, D), jnp.float32),
        ],
    )(a, deg, w0, w1, wout, b, batch2, f1, f1b, f2, f2b, f3, f3b)


def kernel(x, edge_index_dim0, edge_index_dim1, batch,
           W1_d0, W1_d1, W1_out, b1, W2_d0, W2_d1, W2_out, b2,
           F1, f1b, F2, f2b, F3, f3b):
    eis = jnp.concatenate([edge_index_dim0.reshape(-1),
                           edge_index_dim1.reshape(-1)])

    (degw,) = _sc_deg(eis)
    (a1,) = _sc_scatter(x, eis)
    h1 = _tc_conv1(a1, degw, W1_d0, W1_d1, W1_out, b1.reshape(1, D))
    (a2,) = _sc_scatter(h1, eis)
    return _tc_conv2(a2, degw, W2_d0, W2_d1, W2_out, b2.reshape(1, D),
                     batch.reshape(N, 1), F1, f1b.reshape(1, 84),
                     F2, f2b.reshape(1, 42), F3, f3b.reshape(1, D))


# speedup vs baseline: 8.1497x; 1.5924x over previous
import jax
import jax.numpy as jnp
from jax import lax
from jax.experimental import pallas as pl
from jax.experimental.pallas import tpu as pltpu
from jax.experimental.pallas import tpu_sc as plsc

N = 10000
E = 320000
D = 128
G = 64

_NS = 16
_EPT = E // _NS
_C = 128
_NCH = _EPT // _C
_TAIL = _EPT - _NCH * _C
_RPT = 624
_RREM = N - _RPT * _NS

_BLKS = [(0, 128), (128, 128), (256, 128), (384, 128), (496, 128)]


_NCHP = 160
_PCH = 32
_NPH = _NCHP // _PCH
_NPAD = 16
_NP = N + _NPAD


def _make_sc_scatter():
    mesh = plsc.VectorSubcoreMesh(core_axis_name="c", subcore_axis_name="s")
    out_type = [jax.ShapeDtypeStruct((2, N, D), jnp.float32)]
    scratch = [
        pltpu.VMEM((_PCH, _C), jnp.int32),
        pltpu.VMEM((_PCH, _C), jnp.int32),
        pltpu.VMEM((_C, D), jnp.float32),
        pltpu.VMEM((_C, D), jnp.float32),
        pltpu.VMEM_SHARED((_NP, D), jnp.float32),
        pltpu.SemaphoreType.DMA,
        pltpu.SemaphoreType.DMA,
        pltpu.SemaphoreType.DMA,
        pltpu.SemaphoreType.DMA,
    ]

    def body(h_hbm, ei_hbm, acc_out, idx_s, idx_d, rowsA, rowsB, acc,
             gA, gB, sA, sB):
        c = lax.axis_index("c")
        s = lax.axis_index("s")
        r0 = s * _RPT
        rr = _RPT * _NS

        def zrow(r, carry):
            def zcol(k, carry2):
                rowsA[r, pl.ds(k * 16, 16)] = jnp.zeros((16,), jnp.float32)
                return carry2
            lax.fori_loop(0, D // 16, zcol, None)
            return carry

        lax.fori_loop(0, _C, zrow, None)

        for boff, blen in _BLKS:
            pltpu.sync_copy(rowsA.at[pl.ds(0, blen)],
                            acc.at[pl.ds(r0 + boff, blen)])

        @pl.when(s == _NS - 1)
        def _():
            pltpu.sync_copy(rowsA.at[pl.ds(0, _RREM + _NPAD)],
                            acc.at[pl.ds(rr, _RREM + _NPAD)])

        plsc.subcore_barrier()

        def start_g(j, rows, sem):
            pltpu.async_copy(h_hbm.at[idx_s.at[j]], rows, sem)

        def wait_g(j, rows, sem):
            pltpu.make_async_copy(h_hbm.at[idx_s.at[j]], rows, sem).wait()

        def start_s(j, rows, sem):
            pltpu.async_copy(rows, acc.at[idx_d.at[j]], sem, add=True)

        def wait_s(j, rows, sem):
            pltpu.make_async_copy(rows, acc.at[idx_d.at[j]], sem).wait()

        for ph in range(_NPH):
            pltpu.sync_copy(ei_hbm.at[2 * c, s, pl.ds(ph * _PCH, _PCH)], idx_s)
            pltpu.sync_copy(ei_hbm.at[2 * c + 1, s, pl.ds(ph * _PCH, _PCH)], idx_d)
            start_g(0, rowsA, gA)

            def pipe(p, carry):
                a = 2 * p
                start_g(a + 1, rowsB, gB)
                wait_g(a, rowsA, gA)
                start_s(a, rowsA, sA)
                wait_s(a, rowsA, sA)

                @pl.when(p < _PCH // 2 - 1)
                def _():
                    start_g(a + 2, rowsA, gA)

                wait_g(a + 1, rowsB, gB)
                start_s(a + 1, rowsB, sB)
                wait_s(a + 1, rowsB, sB)
                return carry

            lax.fori_loop(0, _PCH // 2, pipe, None)

        plsc.subcore_barrier()
        for boff, blen in _BLKS:
            pltpu.sync_copy(acc.at[pl.ds(r0 + boff, blen)],
                            rowsA.at[pl.ds(0, blen)])
            pltpu.sync_copy(rowsA.at[pl.ds(0, blen)],
                            acc_out.at[c, pl.ds(r0 + boff, blen)])

        @pl.when(s == _NS - 1)
        def _():
            pltpu.sync_copy(acc.at[pl.ds(rr, _RREM)], rowsB.at[pl.ds(0, _RREM)])
            pltpu.sync_copy(rowsB.at[pl.ds(0, _RREM)],
                            acc_out.at[c, pl.ds(rr, _RREM)])

    return pl.kernel(body, out_type=out_type, mesh=mesh, scratch_types=scratch)


def _make_sc_deg():
    mesh = plsc.VectorSubcoreMesh(core_axis_name="c", subcore_axis_name="s")
    out_type = [jax.ShapeDtypeStruct((2, N, D), jnp.float32)]
    scratch = [
        pltpu.VMEM((_C,), jnp.int32),
        pltpu.VMEM((_TAIL,), jnp.int32),
        pltpu.VMEM((_C, D), jnp.float32),
        pltpu.VMEM_SHARED((N, D), jnp.float32),
    ]

    def body(ei_hbm, deg_out, idx_d, tidx_d, ones_v, degacc):
        c = lax.axis_index("c")
        s = lax.axis_index("s")
        r0 = s * _RPT
        rr = _RPT * _NS

        def fill(val):
            def frow(r, carry):
                def fcol(k, carry2):
                    ones_v[r, pl.ds(k * 16, 16)] = jnp.full((16,), val, jnp.float32)
                    return carry2
                lax.fori_loop(0, D // 16, fcol, None)
                return carry
            lax.fori_loop(0, _C, frow, None)

        fill(0.0)
        for boff, blen in _BLKS:
            pltpu.sync_copy(ones_v.at[pl.ds(0, blen)],
                            degacc.at[pl.ds(r0 + boff, blen)])

        @pl.when(s == _NS - 1)
        def _():
            pltpu.sync_copy(ones_v.at[pl.ds(0, _RREM)], degacc.at[pl.ds(rr, _RREM)])

        fill(1.0)
        plsc.subcore_barrier()

        dbase = c * (2 * E) + E + s * _EPT

        def chunk(j, _):
            pltpu.sync_copy(ei_hbm.at[pl.ds(dbase + j * _C, _C)], idx_d)
            pltpu.sync_copy(ones_v, degacc.at[idx_d], add=True)
            return _

        lax.fori_loop(0, _NCH, chunk, None)
        pltpu.sync_copy(ei_hbm.at[pl.ds(dbase + _NCH * _C, _TAIL)], tidx_d)
        pltpu.sync_copy(ones_v.at[pl.ds(0, _TAIL)], degacc.at[tidx_d], add=True)

        plsc.subcore_barrier()
        for boff, blen in _BLKS:
            pltpu.sync_copy(degacc.at[pl.ds(r0 + boff, blen)],
                            ones_v.at[pl.ds(0, blen)])
            pltpu.sync_copy(ones_v.at[pl.ds(0, blen)],
                            deg_out.at[c, pl.ds(r0 + boff, blen)])

        @pl.when(s == _NS - 1)
        def _():
            pltpu.sync_copy(degacc.at[pl.ds(rr, _RREM)], ones_v.at[pl.ds(0, _RREM)])
            pltpu.sync_copy(ones_v.at[pl.ds(0, _RREM)],
                            deg_out.at[c, pl.ds(rr, _RREM)])

    return pl.kernel(body, out_type=out_type, mesh=mesh, scratch_types=scratch)


_sc_scatter = _make_sc_scatter()
_sc_deg = _make_sc_deg()


_R = 1000
_NB = N // _R


def _tc_conv1_body(a_ref, deg_ref, w0_ref, w1_ref, wout_ref, b_ref, out_ref):
    f32 = jnp.float32
    wc0 = jnp.dot(w0_ref[...], wout_ref[...], preferred_element_type=f32)
    wc1 = jnp.dot(w1_ref[...], wout_ref[...], preferred_element_type=f32)
    r0 = 1.0 / jnp.maximum(deg_ref[0][:, :1], 1.0)
    r1 = 1.0 / jnp.maximum(deg_ref[1][:, :1], 1.0)
    out_ref[...] = (jnp.dot(a_ref[0] * r0, wc0, preferred_element_type=f32)
                    + jnp.dot(a_ref[1] * r1, wc1, preferred_element_type=f32)
                    + b_ref[...])


def _tc_conv1(a, deg, w0, w1, wout, b):
    return pl.pallas_call(
        _tc_conv1_body,
        grid=(_NB,),
        in_specs=[
            pl.BlockSpec((2, _R, D), lambda i: (0, i, 0)),
            pl.BlockSpec((2, _R, D), lambda i: (0, i, 0)),
            pl.BlockSpec((D, D), lambda i: (0, 0)),
            pl.BlockSpec((D, D), lambda i: (0, 0)),
            pl.BlockSpec((D, D), lambda i: (0, 0)),
            pl.BlockSpec((1, D), lambda i: (0, 0)),
        ],
        out_specs=pl.BlockSpec((_R, D), lambda i: (i, 0)),
        out_shape=jax.ShapeDtypeStruct((N, D), jnp.float32),
    )(a, deg, w0, w1, wout, b)


def _leaky(v):
    return jnp.where(v > 0, v, 0.01 * v)


def _tc_conv2_body(a_ref, deg_ref, w0_ref, w1_ref, wout_ref, b_ref, batch_ref,
                   f1_ref, f1b_ref, f2_ref, f2b_ref, f3_ref, f3b_ref,
                   out_ref, pooled_s, cnt_s):
    f32 = jnp.float32
    i = pl.program_id(0)
    wc0 = jnp.dot(w0_ref[...], wout_ref[...], preferred_element_type=f32)
    wc1 = jnp.dot(w1_ref[...], wout_ref[...], preferred_element_type=f32)
    r0 = 1.0 / jnp.maximum(deg_ref[0][:, :1], 1.0)
    r1 = 1.0 / jnp.maximum(deg_ref[1][:, :1], 1.0)
    h = (jnp.dot(a_ref[0] * r0, wc0, preferred_element_type=f32)
         + jnp.dot(a_ref[1] * r1, wc1, preferred_element_type=f32)
         + b_ref[...])
    oh = (batch_ref[...] == lax.broadcasted_iota(jnp.int32, (_R, G), 1)).astype(f32)
    dnums = (((0,), (0,)), ((), ()))
    pb = lax.dot_general(oh, h, dnums, preferred_element_type=f32)
    cb = lax.dot_general(oh, jnp.ones_like(h), dnums, preferred_element_type=f32)

    @pl.when(i == 0)
    def _():
        pooled_s[...] = pb
        cnt_s[...] = cb

    @pl.when(i > 0)
    def _():
        pooled_s[...] += pb
        cnt_s[...] += cb

    @pl.when(i == _NB - 1)
    def _():
        ge = pooled_s[...] / jnp.maximum(cnt_s[...], 1.0)
        z = _leaky(jnp.dot(ge, f1_ref[...], preferred_element_type=f32) + f1b_ref[...])
        z = _leaky(jnp.dot(z, f2_ref[...], preferred_element_type=f32) + f2b_ref[...])
        out_ref[...] = jnp.dot(z, f3_ref[...], preferred_element_type=f32) + f3b_ref[...]


def _tc_conv2(a, deg, w0, w1, wout, b, batch2, f1, f1b, f2, f2b, f3, f3b):
    full = lambda shape: pl.BlockSpec(shape, lambda i: tuple(0 for _ in shape))
    return pl.pallas_call(
        _tc_conv2_body,
        grid=(_NB,),
        in_specs=[
            pl.BlockSpec((2, _R, D), lambda i: (0, i, 0)),
            pl.BlockSpec((2, _R, D), lambda i: (0, i, 0)),
            full((D, D)), full((D, D)), full((D, D)), full((1, D)),
            pl.BlockSpec((_R, 1), lambda i: (i, 0)),
            full((D, 84)), full((1, 84)),
            full((84, 42)), full((1, 42)),
            full((42, D)), full((1, D)),
        ],
        out_specs=pl.BlockSpec((G, D), lambda i: (0, 0)),
        out_shape=jax.ShapeDtypeStruct((G, D), jnp.float32),
        scratch_shapes=[
            pltpu.VMEM((G, D), jnp.float32),
            pltpu.VMEM((G, D), jnp.float32),
        ],
    )(a, deg, w0, w1, wout, b, batch2, f1, f1b, f2, f2b, f3, f3b)


def kernel(x, edge_index_dim0, edge_index_dim1, batch,
           W1_d0, W1_d1, W1_out, b1, W2_d0, W2_d1, W2_out, b2,
           F1, f1b, F2, f2b, F3, f3b):
    eis = jnp.concatenate([edge_index_dim0.reshape(-1),
                           edge_index_dim1.reshape(-1)])

    npad = _NCHP * _C - _EPT
    pad_iota = jax.lax.broadcasted_iota(jnp.int32, (_NS, npad), 1)
    spad = pad_iota % 16
    dpad = N + (pad_iota % _NPAD)

    def _pack(src, dst):
        s3 = jnp.concatenate([src.reshape(_NS, _EPT), spad], axis=1)
        d3 = jnp.concatenate([dst.reshape(_NS, _EPT), dpad], axis=1)
        return s3.reshape(_NS, _NCHP, _C), d3.reshape(_NS, _NCHP, _C)

    s0, d0 = _pack(edge_index_dim0[0], edge_index_dim0[1])
    s1, d1 = _pack(edge_index_dim1[0], edge_index_dim1[1])
    eis4 = jnp.stack([s0, d0, s1, d1])

    (degw,) = _sc_deg(eis)
    (a1,) = _sc_scatter(x, eis4)
    h1 = _tc_conv1(a1, degw, W1_d0, W1_d1, W1_out, b1.reshape(1, D))
    (a2,) = _sc_scatter(h1, eis4)
    return _tc_conv2(a2, degw, W2_d0, W2_d1, W2_out, b2.reshape(1, D),
                     batch.reshape(N, 1), F1, f1b.reshape(1, 84),
                     F2, f2b.reshape(1, 42), F3, f3b.reshape(1, D))
